# Initial kernel scaffold; baseline (speedup 1.0000x reference)
import jax
import jax.numpy as jnp
from jax.experimental import pallas as pl


def kernel(prototype_activations, upsampled_activation, W, b):
    raise NotImplementedError("write your pallas kernel here")


# speedup vs baseline: 13.6165x; 13.6165x over previous
import functools

import jax
import jax.numpy as jnp
from jax import lax
from jax.experimental import pallas as pl
from jax.experimental.pallas import tpu as pltpu
from jax.experimental.pallas import tpu_sc as plsc

NUM_CORES = 2
NUM_SUBCORES = 16
NUM_WORKERS = NUM_CORES * NUM_SUBCORES
LANES = 16

TOPK = 5
NEG = float("-inf")


def _sc_body(acts, out, buf, simbuf, *, rpw, blk, hw, unroll):
    nblk = rpw // blk
    nvec = hw // LANES
    wid = lax.axis_index("s") * NUM_CORES + lax.axis_index("c")
    base = wid * rpw
    lane = lax.iota(jnp.int32, LANES)
    ones = jnp.ones((LANES,), jnp.float32)

    def block_body(bi, carry):
        pltpu.sync_copy(acts.at[pl.ds(base + bi * blk, blk)], buf)
        for r in range(blk):
            neg = jnp.full((LANES,), NEG, jnp.float32)
            init = (neg, neg, neg, neg, neg)

            def vec_body(i, v):
                v1, v2, v3, v4, v5 = v
                for j in range(unroll):
                    x = buf[r, pl.ds((i * unroll + j) * LANES, LANES)]
                    t = jnp.maximum(v1, x); x = jnp.minimum(v1, x); v1 = t
                    t = jnp.maximum(v2, x); x = jnp.minimum(v2, x); v2 = t
                    t = jnp.maximum(v3, x); x = jnp.minimum(v3, x); v3 = t
                    t = jnp.maximum(v4, x); x = jnp.minimum(v4, x); v4 = t
                    v5 = jnp.maximum(v5, x)
                return (v1, v2, v3, v4, v5)

            v1, v2, v3, v4, v5 = lax.fori_loop(0, nvec // unroll, vec_body, init)

            s = jnp.float32(0.0)
            for _ in range(TOPK):
                m = jnp.max(v1)
                s = s + m
                f = plsc.all_reduce_ffs(v1 == m)
                msk = lane == f
                v1 = jnp.where(msk, v2, v1)
                v2 = jnp.where(msk, v3, v2)
                v3 = jnp.where(msk, v4, v3)
                v4 = jnp.where(msk, v5, v4)
            sim = s * jnp.float32(1.0 / TOPK)

            idx = jnp.full((LANES,), bi * blk + r, jnp.int32)
            plsc.store_scatter(simbuf, [idx], ones * sim, mask=lane == 0)
        return carry

    lax.fori_loop(0, nblk, block_body, 0)
    pltpu.sync_copy(simbuf, out.at[wid])


def _build_sc(nrows, hw, blk, pad, unroll, interpret=False):
    rpw = nrows // NUM_WORKERS
    mesh = plsc.VectorSubcoreMesh(
        core_axis_name="c", subcore_axis_name="s",
        num_cores=NUM_CORES, num_subcores=NUM_SUBCORES)
    return pl.kernel(
        functools.partial(_sc_body, rpw=rpw, blk=blk, hw=hw, unroll=unroll),
        out_type=jax.ShapeDtypeStruct((NUM_WORKERS, pad), jnp.float32),
        mesh=mesh,
        scratch_types=[
            pltpu.VMEM((blk, hw), jnp.float32),
            pltpu.VMEM((pad,), jnp.float32),
        ],
        compiler_params=pltpu.CompilerParams(needs_layout_passes=False),
        interpret=interpret,
    )


def _tc_linear(sim_ref, w_ref, b_ref, o_ref):
    s = sim_ref[...].astype(jnp.bfloat16).astype(jnp.float32)
    w = w_ref[...].astype(jnp.bfloat16).astype(jnp.float32)
    o_ref[...] = jnp.sum(s * w, axis=1, keepdims=True) + b_ref[...]


def kernel(prototype_activations, upsampled_activation, W, b):
    B, P = prototype_activations.shape[0], prototype_activations.shape[1]
    hw = prototype_activations.shape[2] * prototype_activations.shape[3]
    nrows = B * P
    rpw = nrows // NUM_WORKERS
    pad = (rpw + LANES - 1) // LANES * LANES
    acts = prototype_activations.reshape(nrows, hw)

    sc = _build_sc(nrows, hw, blk=8, pad=pad, unroll=4)
    simp = sc(acts)
    sim = simp[:, :rpw].reshape(B, P)

    logits = pl.pallas_call(
        _tc_linear,
        out_shape=jax.ShapeDtypeStruct((B, 1), jnp.float32),
    )(sim, W, b.reshape(1, 1))
    return logits


# speedup vs baseline: 15.4881x; 1.1375x over previous
import functools

import jax
import jax.numpy as jnp
from jax import lax
from jax.experimental import pallas as pl
from jax.experimental.pallas import tpu as pltpu
from jax.experimental.pallas import tpu_sc as plsc

NUM_CORES = 2
NUM_SUBCORES = 16
NUM_WORKERS = NUM_CORES * NUM_SUBCORES
LANES = 16

TOPK = 5
NEG = float("-inf")


def _sc_body(acts, out, buf, simbuf, *, bpw, p_dim, sub, blk, pad, unroll):
    nblk = p_dim // blk
    wid = lax.axis_index("s") * NUM_CORES + lax.axis_index("c")
    b0 = wid * bpw
    lane = lax.iota(jnp.int32, LANES)
    ones = jnp.ones((LANES,), jnp.float32)

    def block_body(t, carry):
        bb = t // nblk
        bi = t % nblk
        pltpu.sync_copy(acts.at[b0 + bb, pl.ds(bi * blk, blk)], buf)
        for r in range(blk):
            neg = jnp.full((LANES,), NEG, jnp.float32)
            init = (neg, neg, neg, neg, neg)

            def vec_body(i, v):
                v1, v2, v3, v4, v5 = v
                for j in range(unroll):
                    x = buf[r, i, pl.ds(j * LANES, LANES)]
                    t1 = jnp.maximum(v1, x); x = jnp.minimum(v1, x); v1 = t1
                    t1 = jnp.maximum(v2, x); x = jnp.minimum(v2, x); v2 = t1
                    t1 = jnp.maximum(v3, x); x = jnp.minimum(v3, x); v3 = t1
                    t1 = jnp.maximum(v4, x); x = jnp.minimum(v4, x); v4 = t1
                    v5 = jnp.maximum(v5, x)
                return (v1, v2, v3, v4, v5)

            v1, v2, v3, v4, v5 = lax.fori_loop(0, sub, vec_body, init)

            s = jnp.float32(0.0)
            for _ in range(TOPK):
                m = jnp.max(v1)
                s = s + m
                f = plsc.all_reduce_ffs(v1 == m)
                msk = lane == f
                v1 = jnp.where(msk, v2, v1)
                v2 = jnp.where(msk, v3, v2)
                v3 = jnp.where(msk, v4, v3)
                v4 = jnp.where(msk, v5, v4)
            sim = s * jnp.float32(1.0 / TOPK)

            idx = jnp.full((LANES,), bb * p_dim + bi * blk + r, jnp.int32)
            plsc.store_scatter(simbuf, [idx], ones * sim, mask=lane == 0)
        return carry

    lax.fori_loop(0, bpw * nblk, block_body, 0)
    pltpu.sync_copy(simbuf, out.at[wid])


def _build_sc(bdim, p_dim, sub, lanes, blk, pad, interpret=False):
    bpw = bdim // NUM_WORKERS
    unroll = lanes // LANES
    mesh = plsc.VectorSubcoreMesh(
        core_axis_name="c", subcore_axis_name="s",
        num_cores=NUM_CORES, num_subcores=NUM_SUBCORES)
    return pl.kernel(
        functools.partial(_sc_body, bpw=bpw, p_dim=p_dim, sub=sub, blk=blk,
                          pad=pad, unroll=unroll),
        out_type=jax.ShapeDtypeStruct((NUM_WORKERS, pad), jnp.float32),
        mesh=mesh,
        scratch_types=[
            pltpu.VMEM((blk, sub, lanes), jnp.float32),
            pltpu.VMEM((pad,), jnp.float32),
        ],
        compiler_params=pltpu.CompilerParams(needs_layout_passes=False),
        interpret=interpret,
    )


def _tc_linear(sim_ref, w_ref, b_ref, o_ref):
    s = sim_ref[...].astype(jnp.bfloat16).astype(jnp.float32)
    w = w_ref[...].astype(jnp.bfloat16).astype(jnp.float32)
    o_ref[...] = jnp.sum(s * w, axis=1, keepdims=True) + b_ref[...]


def kernel(prototype_activations, upsampled_activation, W, b):
    B, P = prototype_activations.shape[0], prototype_activations.shape[1]
    sub, lanes = prototype_activations.shape[2], prototype_activations.shape[3]
    rpw = (B // NUM_WORKERS) * P
    pad = (rpw + LANES - 1) // LANES * LANES

    sc = _build_sc(B, P, sub, lanes, blk=10, pad=pad)
    simp = sc(prototype_activations)
    sim = simp[:, :rpw].reshape(B, P)

    logits = pl.pallas_call(
        _tc_linear,
        out_shape=jax.ShapeDtypeStruct((B, 1), jnp.float32),
    )(sim, W, b.reshape(1, 1))
    return logits
